# baseline (speedup 1.0000x reference)
import functools

import jax
import jax.numpy as jnp
from jax import lax
from jax.experimental import pallas as pl
from jax.experimental.pallas import tpu as pltpu
from jax.experimental.pallas import tpu_sc as plsc

SCORE_THRESH = 0.05
NMS_THRESH = 0.5
MAX_DET = 100
L = 16
NT = 16
NEG = float("-inf")


def _make_nms(n_pad: int):
    per = n_pad // NT
    chunks = per // L
    mesh = plsc.VectorSubcoreMesh(core_axis_name="c", subcore_axis_name="s", num_cores=1)

    @functools.partial(
        pl.kernel,
        mesh=mesh,
        compiler_params=pltpu.CompilerParams(needs_layout_passes=False),
        out_type=jax.ShapeDtypeStruct((MAX_DET * L,), jnp.float32),
        scratch_types=[
            pltpu.VMEM((per,), jnp.float32),
            pltpu.VMEM((per,), jnp.float32),
            pltpu.VMEM((per,), jnp.float32),
            pltpu.VMEM((per,), jnp.float32),
            pltpu.VMEM((per,), jnp.float32),
            pltpu.VMEM((per,), jnp.float32),
            pltpu.VMEM((L,), jnp.float32),
            pltpu.VMEM((NT, 128), jnp.float32),
            pltpu.VMEM((MAX_DET * L,), jnp.float32),
            pltpu.VMEM_SHARED((2, NT, 128), jnp.float32),
        ],
    )
    def nms(x1_hbm, y1_hbm, x2_hbm, y2_hbm, sc_hbm, out_hbm,
            x1v, y1v, x2v, y2v, sv, areav, candv, allv, outv, shared):
        cid = lax.axis_index("c")
        sid = lax.axis_index("s")
        base = sid * per

        pltpu.sync_copy(x1_hbm.at[pl.ds(base, per)], x1v)
        pltpu.sync_copy(y1_hbm.at[pl.ds(base, per)], y1v)
        pltpu.sync_copy(x2_hbm.at[pl.ds(base, per)], x2v)
        pltpu.sync_copy(y2_hbm.at[pl.ds(base, per)], y2v)
        pltpu.sync_copy(sc_hbm.at[pl.ds(base, per)], sv)

        lane = lax.iota(jnp.int32, L)

        @plsc.parallel_loop(0, per, step=L, unroll=4)
        def _prep(o):
            sl = pl.ds(o, L)
            a = (x2v[sl] - x1v[sl]) * (y2v[sl] - y1v[sl])
            areav[sl] = a
            s = sv[sl]
            sv[sl] = jnp.where(s > SCORE_THRESH, s, NEG)

        big = jnp.int32(2 ** 30)

        def fused_pass(wx1, wy1, wx2, wy2):
            warea = (wx2 - wx1) * (wy2 - wy1)

            mv0 = jnp.full((L,), NEG, jnp.float32)
            mi0 = jnp.zeros((L,), jnp.int32)

            @plsc.parallel_loop(0, per, step=L, unroll=8, carry=(mv0, mi0))
            def body(o, carry):
                mv, mi = carry
                sl = pl.ds(o, L)
                bx1 = x1v[sl]
                by1 = y1v[sl]
                bx2 = x2v[sl]
                by2 = y2v[sl]
                ltx = jnp.maximum(bx1, wx1)
                lty = jnp.maximum(by1, wy1)
                rbx = jnp.minimum(bx2, wx2)
                rby = jnp.minimum(by2, wy2)
                w = jnp.maximum(rbx - ltx, 0.0)
                h = jnp.maximum(rby - lty, 0.0)
                inter = w * h
                iou = inter / (areav[sl] + warea - inter + 1e-9)
                ns = jnp.where(iou > NMS_THRESH, NEG, sv[sl])
                sv[sl] = ns
                idx = jnp.full((L,), o, jnp.int32) + lane
                gt = ns > mv
                mv = jnp.where(gt, ns, mv)
                mi = jnp.where(gt, idx, mi)
                return (mv, mi)

            mv, mi = body
            m = jnp.max(mv)
            cand = jnp.where(mv == m, mi, big)
            li = jnp.min(cand)
            return m, li

        m0, li0 = fused_pass(
            jnp.full((L,), 1e9, jnp.float32),
            jnp.full((L,), 1e9, jnp.float32),
            jnp.full((L,), -1e9, jnp.float32),
            jnp.full((L,), -1e9, jnp.float32),
        )

        zero = jnp.zeros((L,), jnp.int32)

        def round_body(i, carry):
            m, li = carry
            liv = jnp.full((L,), li, jnp.int32)
            cx1 = plsc.load_gather(x1v, [liv])
            cy1 = plsc.load_gather(y1v, [liv])
            cx2 = plsc.load_gather(x2v, [liv])
            cy2 = plsc.load_gather(y2v, [liv])
            mvv = jnp.full((L,), m, jnp.float32)
            row = jnp.where(lane == 0, mvv,
                  jnp.where(lane == 1, cx1,
                  jnp.where(lane == 2, cy1,
                  jnp.where(lane == 3, cx2,
                  jnp.where(lane == 4, cy2, jnp.float32(0.0))))))
            candv[...] = row
            p = lax.rem(i, 2)
            pltpu.sync_copy(candv, shared.at[p, sid, pl.ds(0, L)])
            plsc.subcore_barrier()
            pltpu.sync_copy(shared.at[p], allv)

            vals = plsc.load_gather(allv, [lane, zero])
            gm = jnp.max(vals)
            t = plsc.all_reduce_ffs(vals == gm)
            tv = jnp.broadcast_to(t, (L,)).astype(jnp.int32)
            wx1 = plsc.load_gather(allv, [tv, jnp.full((L,), 1, jnp.int32)])
            wy1 = plsc.load_gather(allv, [tv, jnp.full((L,), 2, jnp.int32)])
            wx2 = plsc.load_gather(allv, [tv, jnp.full((L,), 3, jnp.int32)])
            wy2 = plsc.load_gather(allv, [tv, jnp.full((L,), 4, jnp.int32)])

            valid = gm > NEG
            gmv = jnp.full((L,), gm, jnp.float32)
            orow = jnp.where(lane == 0, wx1,
                   jnp.where(lane == 1, wy1,
                   jnp.where(lane == 2, wx2,
                   jnp.where(lane == 3, wy2,
                   jnp.where(lane == 4, gmv, jnp.float32(0.0))))))
            orow = jnp.where((lane < 5) & valid, orow, jnp.float32(0.0))
            outv[pl.ds(i * L, L)] = orow

            return fused_pass(wx1, wy1, wx2, wy2)

        lax.fori_loop(0, MAX_DET, round_body, (m0, li0))

        @pl.when(jnp.logical_and(cid == 0, sid == 0))
        def _():
            pltpu.sync_copy(outv, out_hbm)

    return nms


def kernel(boxes, scores):
    n = boxes.shape[0]
    n_pad = ((n + NT * L - 1) // (NT * L)) * (NT * L)
    x1 = jnp.zeros((n_pad,), jnp.float32).at[:n].set(boxes[:, 0])
    y1 = jnp.zeros((n_pad,), jnp.float32).at[:n].set(boxes[:, 1])
    x2 = jnp.zeros((n_pad,), jnp.float32).at[:n].set(boxes[:, 2])
    y2 = jnp.zeros((n_pad,), jnp.float32).at[:n].set(boxes[:, 3])
    sc = jnp.zeros((n_pad,), jnp.float32).at[:n].set(scores)
    out = _make_nms(n_pad)(x1, y1, x2, y2, sc)
    return out.reshape(MAX_DET, L)[:, :5]


# speedup vs baseline: 1.1039x; 1.1039x over previous
import functools

import jax
import jax.numpy as jnp
from jax import lax
from jax.experimental import pallas as pl
from jax.experimental.pallas import tpu as pltpu
from jax.experimental.pallas import tpu_sc as plsc

SCORE_THRESH = 0.05
NMS_THRESH = 0.5
MAX_DET = 100
L = 16
NT = 16
NEG = float("-inf")


def _make_nms(n_pad: int):
    per = n_pad // NT
    chunks = per // L
    mesh = plsc.VectorSubcoreMesh(core_axis_name="c", subcore_axis_name="s", num_cores=1)

    @functools.partial(
        pl.kernel,
        mesh=mesh,
        compiler_params=pltpu.CompilerParams(needs_layout_passes=False),
        out_type=jax.ShapeDtypeStruct((MAX_DET * L,), jnp.float32),
        scratch_types=[
            pltpu.VMEM((per,), jnp.float32),
            pltpu.VMEM((per,), jnp.float32),
            pltpu.VMEM((per,), jnp.float32),
            pltpu.VMEM((per,), jnp.float32),
            pltpu.VMEM((per,), jnp.float32),
            pltpu.VMEM((per,), jnp.float32),
            pltpu.VMEM((L,), jnp.float32),
            pltpu.VMEM((NT, 128), jnp.float32),
            pltpu.VMEM((MAX_DET * L,), jnp.float32),
            pltpu.VMEM_SHARED((2, NT, 128), jnp.float32),
        ],
    )
    def nms(x1_hbm, y1_hbm, x2_hbm, y2_hbm, sc_hbm, out_hbm,
            x1v, y1v, x2v, y2v, sv, areav, candv, allv, outv, shared):
        cid = lax.axis_index("c")
        sid = lax.axis_index("s")
        base = sid * per

        pltpu.sync_copy(x1_hbm.at[pl.ds(base, per)], x1v)
        pltpu.sync_copy(y1_hbm.at[pl.ds(base, per)], y1v)
        pltpu.sync_copy(x2_hbm.at[pl.ds(base, per)], x2v)
        pltpu.sync_copy(y2_hbm.at[pl.ds(base, per)], y2v)
        pltpu.sync_copy(sc_hbm.at[pl.ds(base, per)], sv)

        lane = lax.iota(jnp.int32, L)

        @plsc.parallel_loop(0, per, step=L, unroll=4)
        def _prep(o):
            sl = pl.ds(o, L)
            a = (x2v[sl] - x1v[sl]) * (y2v[sl] - y1v[sl])
            areav[sl] = a
            s = sv[sl]
            sv[sl] = jnp.where(s > SCORE_THRESH, s, NEG)

        big = jnp.int32(2 ** 30)

        def fused_pass(wx1, wy1, wx2, wy2):
            warea = (wx2 - wx1) * (wy2 - wy1)

            mv0 = jnp.full((L,), NEG, jnp.float32)
            mi0 = jnp.zeros((L,), jnp.int32)

            @plsc.parallel_loop(0, per, step=L, unroll=2, carry=(mv0, mi0))
            def body(o, carry):
                mv, mi = carry
                sl = pl.ds(o, L)
                bx1 = x1v[sl]
                by1 = y1v[sl]
                bx2 = x2v[sl]
                by2 = y2v[sl]
                ltx = jnp.maximum(bx1, wx1)
                lty = jnp.maximum(by1, wy1)
                rbx = jnp.minimum(bx2, wx2)
                rby = jnp.minimum(by2, wy2)
                w = jnp.maximum(rbx - ltx, 0.0)
                h = jnp.maximum(rby - lty, 0.0)
                inter = w * h
                iou = inter / (areav[sl] + warea - inter + 1e-9)
                ns = jnp.where(iou > NMS_THRESH, NEG, sv[sl])
                sv[sl] = ns
                idx = jnp.full((L,), o, jnp.int32) + lane
                gt = ns > mv
                mv = jnp.where(gt, ns, mv)
                mi = jnp.where(gt, idx, mi)
                return (mv, mi)

            mv, mi = body
            m = jnp.max(mv)
            cand = jnp.where(mv == m, mi, big)
            li = jnp.min(cand)
            return m, li

        m0, li0 = fused_pass(
            jnp.full((L,), 1e9, jnp.float32),
            jnp.full((L,), 1e9, jnp.float32),
            jnp.full((L,), -1e9, jnp.float32),
            jnp.full((L,), -1e9, jnp.float32),
        )

        zero = jnp.zeros((L,), jnp.int32)

        def round_body(i, carry):
            m, li = carry
            liv = jnp.full((L,), li, jnp.int32)
            cx1 = plsc.load_gather(x1v, [liv])
            cy1 = plsc.load_gather(y1v, [liv])
            cx2 = plsc.load_gather(x2v, [liv])
            cy2 = plsc.load_gather(y2v, [liv])
            mvv = jnp.full((L,), m, jnp.float32)
            row = jnp.where(lane == 0, mvv,
                  jnp.where(lane == 1, cx1,
                  jnp.where(lane == 2, cy1,
                  jnp.where(lane == 3, cx2,
                  jnp.where(lane == 4, cy2, jnp.float32(0.0))))))
            candv[...] = row
            p = lax.rem(i, 2)
            pltpu.sync_copy(candv, shared.at[p, sid, pl.ds(0, L)])
            plsc.subcore_barrier()
            pltpu.sync_copy(shared.at[p], allv)

            vals = plsc.load_gather(allv, [lane, zero])
            gm = jnp.max(vals)
            t = plsc.all_reduce_ffs(vals == gm)
            tv = jnp.broadcast_to(t, (L,)).astype(jnp.int32)
            wx1 = plsc.load_gather(allv, [tv, jnp.full((L,), 1, jnp.int32)])
            wy1 = plsc.load_gather(allv, [tv, jnp.full((L,), 2, jnp.int32)])
            wx2 = plsc.load_gather(allv, [tv, jnp.full((L,), 3, jnp.int32)])
            wy2 = plsc.load_gather(allv, [tv, jnp.full((L,), 4, jnp.int32)])

            valid = gm > NEG
            gmv = jnp.full((L,), gm, jnp.float32)
            orow = jnp.where(lane == 0, wx1,
                   jnp.where(lane == 1, wy1,
                   jnp.where(lane == 2, wx2,
                   jnp.where(lane == 3, wy2,
                   jnp.where(lane == 4, gmv, jnp.float32(0.0))))))
            orow = jnp.where((lane < 5) & valid, orow, jnp.float32(0.0))
            outv[pl.ds(i * L, L)] = orow

            return fused_pass(wx1, wy1, wx2, wy2)

        lax.fori_loop(0, MAX_DET, round_body, (m0, li0))

        @pl.when(jnp.logical_and(cid == 0, sid == 0))
        def _():
            pltpu.sync_copy(outv, out_hbm)

    return nms


def kernel(boxes, scores):
    n = boxes.shape[0]
    n_pad = ((n + NT * L - 1) // (NT * L)) * (NT * L)
    x1 = jnp.zeros((n_pad,), jnp.float32).at[:n].set(boxes[:, 0])
    y1 = jnp.zeros((n_pad,), jnp.float32).at[:n].set(boxes[:, 1])
    x2 = jnp.zeros((n_pad,), jnp.float32).at[:n].set(boxes[:, 2])
    y2 = jnp.zeros((n_pad,), jnp.float32).at[:n].set(boxes[:, 3])
    sc = jnp.zeros((n_pad,), jnp.float32).at[:n].set(scores)
    out = _make_nms(n_pad)(x1, y1, x2, y2, sc)
    return out.reshape(MAX_DET, L)[:, :5]


# speedup vs baseline: 1.2251x; 1.1097x over previous
import functools

import jax
import jax.numpy as jnp
from jax import lax
from jax.experimental import pallas as pl
from jax.experimental.pallas import tpu as pltpu
from jax.experimental.pallas import tpu_sc as plsc

SCORE_THRESH = 0.05
NMS_THRESH = 0.5
MAX_DET = 100
L = 16
NT = 16
NEG = float("-inf")


def _make_nms(n_pad: int):
    per = n_pad // NT
    chunks = per // L
    mesh = plsc.VectorSubcoreMesh(core_axis_name="c", subcore_axis_name="s", num_cores=1)

    @functools.partial(
        pl.kernel,
        mesh=mesh,
        compiler_params=pltpu.CompilerParams(needs_layout_passes=False),
        out_type=jax.ShapeDtypeStruct((MAX_DET * L,), jnp.float32),
        scratch_types=[
            pltpu.VMEM((per,), jnp.float32),
            pltpu.VMEM((per,), jnp.float32),
            pltpu.VMEM((per,), jnp.float32),
            pltpu.VMEM((per,), jnp.float32),
            pltpu.VMEM((per,), jnp.float32),
            pltpu.VMEM((per,), jnp.float32),
            pltpu.VMEM((L,), jnp.float32),
            pltpu.VMEM((NT * L,), jnp.float32),
            pltpu.VMEM((MAX_DET * L,), jnp.float32),
            pltpu.VMEM_SHARED((2 * NT * L,), jnp.float32),
        ],
    )
    def nms(x1_hbm, y1_hbm, x2_hbm, y2_hbm, sc_hbm, out_hbm,
            x1v, y1v, x2v, y2v, sv, areav, candv, allv, outv, shared):
        cid = lax.axis_index("c")
        sid = lax.axis_index("s")
        base = sid * per

        pltpu.sync_copy(x1_hbm.at[pl.ds(base, per)], x1v)
        pltpu.sync_copy(y1_hbm.at[pl.ds(base, per)], y1v)
        pltpu.sync_copy(x2_hbm.at[pl.ds(base, per)], x2v)
        pltpu.sync_copy(y2_hbm.at[pl.ds(base, per)], y2v)
        pltpu.sync_copy(sc_hbm.at[pl.ds(base, per)], sv)

        lane = lax.iota(jnp.int32, L)

        @plsc.parallel_loop(0, per, step=L, unroll=4)
        def _prep(o):
            sl = pl.ds(o, L)
            a = (x2v[sl] - x1v[sl]) * (y2v[sl] - y1v[sl])
            areav[sl] = a
            s = sv[sl]
            sv[sl] = jnp.where(s > SCORE_THRESH, s, NEG)

        big = jnp.int32(2 ** 30)

        def fused_pass(wx1, wy1, wx2, wy2):
            warea = (wx2 - wx1) * (wy2 - wy1)

            mv0 = jnp.full((L,), NEG, jnp.float32)
            mi0 = jnp.zeros((L,), jnp.int32)

            @plsc.parallel_loop(0, per, step=L, unroll=2, carry=(mv0, mi0))
            def body(o, carry):
                mv, mi = carry
                sl = pl.ds(o, L)
                bx1 = x1v[sl]
                by1 = y1v[sl]
                bx2 = x2v[sl]
                by2 = y2v[sl]
                ltx = jnp.maximum(bx1, wx1)
                lty = jnp.maximum(by1, wy1)
                rbx = jnp.minimum(bx2, wx2)
                rby = jnp.minimum(by2, wy2)
                w = jnp.maximum(rbx - ltx, 0.0)
                h = jnp.maximum(rby - lty, 0.0)
                inter = w * h
                iou = inter / (areav[sl] + warea - inter + 1e-9)
                ns = jnp.where(iou > NMS_THRESH, NEG, sv[sl])
                sv[sl] = ns
                gt = ns > mv
                mv = jnp.where(gt, ns, mv)
                mi = jnp.where(gt, jnp.full((L,), o, jnp.int32), mi)
                return (mv, mi)

            mv, mi = body
            m = jnp.max(mv)
            cand = jnp.where(mv == m, mi + lane, big)
            li = jnp.min(cand)
            return m, li

        m0, li0 = fused_pass(
            jnp.full((L,), 1e9, jnp.float32),
            jnp.full((L,), 1e9, jnp.float32),
            jnp.full((L,), -1e9, jnp.float32),
            jnp.full((L,), -1e9, jnp.float32),
        )

        zero = jnp.zeros((L,), jnp.int32)

        def round_body(i, carry):
            m, li = carry
            liv = jnp.full((L,), li, jnp.int32)
            cx1 = plsc.load_gather(x1v, [liv])
            cy1 = plsc.load_gather(y1v, [liv])
            cx2 = plsc.load_gather(x2v, [liv])
            cy2 = plsc.load_gather(y2v, [liv])
            mvv = jnp.full((L,), m, jnp.float32)
            row = jnp.where(lane == 0, mvv,
                  jnp.where(lane == 1, cx1,
                  jnp.where(lane == 2, cy1,
                  jnp.where(lane == 3, cx2,
                  jnp.where(lane == 4, cy2, jnp.float32(0.0))))))
            candv[...] = row
            p = lax.rem(i, 2) * (NT * L)
            pltpu.sync_copy(candv, shared.at[pl.ds(p + sid * L, L)])
            plsc.subcore_barrier()
            pltpu.sync_copy(shared.at[pl.ds(p, NT * L)], allv)

            vals = plsc.load_gather(allv, [lane * L])
            gm = jnp.max(vals)
            t = plsc.all_reduce_ffs(vals == gm)
            tb = jnp.broadcast_to(t, (L,)).astype(jnp.int32) * L
            wx1 = plsc.load_gather(allv, [tb + 1])
            wy1 = plsc.load_gather(allv, [tb + 2])
            wx2 = plsc.load_gather(allv, [tb + 3])
            wy2 = plsc.load_gather(allv, [tb + 4])

            valid = gm > NEG
            gmv = jnp.full((L,), gm, jnp.float32)
            orow = jnp.where(lane == 0, wx1,
                   jnp.where(lane == 1, wy1,
                   jnp.where(lane == 2, wx2,
                   jnp.where(lane == 3, wy2,
                   jnp.where(lane == 4, gmv, jnp.float32(0.0))))))
            orow = jnp.where((lane < 5) & valid, orow, jnp.float32(0.0))
            outv[pl.ds(i * L, L)] = orow

            return fused_pass(wx1, wy1, wx2, wy2)

        lax.fori_loop(0, MAX_DET, round_body, (m0, li0))

        @pl.when(jnp.logical_and(cid == 0, sid == 0))
        def _():
            pltpu.sync_copy(outv, out_hbm)

    return nms


def kernel(boxes, scores):
    n = boxes.shape[0]
    n_pad = ((n + NT * L - 1) // (NT * L)) * (NT * L)
    x1 = jnp.zeros((n_pad,), jnp.float32).at[:n].set(boxes[:, 0])
    y1 = jnp.zeros((n_pad,), jnp.float32).at[:n].set(boxes[:, 1])
    x2 = jnp.zeros((n_pad,), jnp.float32).at[:n].set(boxes[:, 2])
    y2 = jnp.zeros((n_pad,), jnp.float32).at[:n].set(boxes[:, 3])
    sc = jnp.zeros((n_pad,), jnp.float32).at[:n].set(scores)
    out = _make_nms(n_pad)(x1, y1, x2, y2, sc)
    return out.reshape(MAX_DET, L)[:, :5]


# speedup vs baseline: 1.2559x; 1.0252x over previous
import functools

import jax
import jax.numpy as jnp
from jax import lax
from jax.experimental import pallas as pl
from jax.experimental.pallas import tpu as pltpu
from jax.experimental.pallas import tpu_sc as plsc

SCORE_THRESH = 0.05
NMS_THRESH = 0.5
MAX_DET = 100
L = 16
NT = 16
NEG = float("-inf")


def _make_nms(n_pad: int):
    per = n_pad // NT
    chunks = per // L
    mesh = plsc.VectorSubcoreMesh(core_axis_name="c", subcore_axis_name="s", num_cores=1)

    @functools.partial(
        pl.kernel,
        mesh=mesh,
        compiler_params=pltpu.CompilerParams(needs_layout_passes=False),
        out_type=jax.ShapeDtypeStruct((MAX_DET * L,), jnp.float32),
        scratch_types=[
            pltpu.VMEM((per,), jnp.float32),
            pltpu.VMEM((per,), jnp.float32),
            pltpu.VMEM((per,), jnp.float32),
            pltpu.VMEM((per,), jnp.float32),
            pltpu.VMEM((per,), jnp.float32),
            pltpu.VMEM((per,), jnp.float32),
            pltpu.VMEM((L,), jnp.float32),
            pltpu.VMEM((NT * L,), jnp.float32),
            pltpu.VMEM((MAX_DET * L,), jnp.float32),
            pltpu.VMEM_SHARED((2 * NT * L,), jnp.float32),
            pltpu.SemaphoreType.DMA,
        ],
    )
    def nms(x1_hbm, y1_hbm, x2_hbm, y2_hbm, sc_hbm, out_hbm,
            x1v, y1v, x2v, y2v, sv, areav, candv, allv, outv, shared, sem):
        cid = lax.axis_index("c")
        sid = lax.axis_index("s")
        base = sid * per

        cps = [pltpu.make_async_copy(src.at[pl.ds(base, per)], dst, sem)
               for src, dst in ((x1_hbm, x1v), (y1_hbm, y1v), (x2_hbm, x2v),
                                (y2_hbm, y2v), (sc_hbm, sv))]
        for cp in cps:
            cp.start()
        for cp in cps:
            cp.wait()

        lane = lax.iota(jnp.int32, L)

        @plsc.parallel_loop(0, per, step=L, unroll=4)
        def _prep(o):
            sl = pl.ds(o, L)
            a = (x2v[sl] - x1v[sl]) * (y2v[sl] - y1v[sl])
            areav[sl] = a
            s = sv[sl]
            sv[sl] = jnp.where(s > SCORE_THRESH, s, NEG)

        big = jnp.int32(2 ** 30)

        def fused_pass(wx1, wy1, wx2, wy2):
            warea = (wx2 - wx1) * (wy2 - wy1)

            mv0 = jnp.full((L,), NEG, jnp.float32)
            mi0 = jnp.zeros((L,), jnp.int32)

            @plsc.parallel_loop(0, per, step=L, unroll=2, carry=(mv0, mi0))
            def body(o, carry):
                mv, mi = carry
                sl = pl.ds(o, L)
                bx1 = x1v[sl]
                by1 = y1v[sl]
                bx2 = x2v[sl]
                by2 = y2v[sl]
                ltx = jnp.maximum(bx1, wx1)
                lty = jnp.maximum(by1, wy1)
                rbx = jnp.minimum(bx2, wx2)
                rby = jnp.minimum(by2, wy2)
                w = jnp.maximum(rbx - ltx, 0.0)
                h = jnp.maximum(rby - lty, 0.0)
                inter = w * h
                iou = inter / (areav[sl] + warea - inter + 1e-9)
                ns = jnp.where(iou > NMS_THRESH, NEG, sv[sl])
                sv[sl] = ns
                gt = ns > mv
                mv = jnp.where(gt, ns, mv)
                mi = jnp.where(gt, jnp.full((L,), o, jnp.int32), mi)
                return (mv, mi)

            mv, mi = body
            m = jnp.max(mv)
            cand = jnp.where(mv == m, mi + lane, big)
            li = jnp.min(cand)
            return m, li

        m0, li0 = fused_pass(
            jnp.full((L,), 1e9, jnp.float32),
            jnp.full((L,), 1e9, jnp.float32),
            jnp.full((L,), -1e9, jnp.float32),
            jnp.full((L,), -1e9, jnp.float32),
        )

        zero = jnp.zeros((L,), jnp.int32)

        def round_body(i, carry):
            m, li = carry
            liv = jnp.full((L,), li, jnp.int32)
            cx1 = plsc.load_gather(x1v, [liv])
            cy1 = plsc.load_gather(y1v, [liv])
            cx2 = plsc.load_gather(x2v, [liv])
            cy2 = plsc.load_gather(y2v, [liv])
            mvv = jnp.full((L,), m, jnp.float32)
            row = jnp.where(lane == 0, mvv,
                  jnp.where(lane == 1, cx1,
                  jnp.where(lane == 2, cy1,
                  jnp.where(lane == 3, cx2,
                  jnp.where(lane == 4, cy2, jnp.float32(0.0))))))
            candv[...] = row
            p = lax.rem(i, 2) * (NT * L)
            pltpu.sync_copy(candv, shared.at[pl.ds(p + sid * L, L)])
            plsc.subcore_barrier()
            pltpu.sync_copy(shared.at[pl.ds(p, NT * L)], allv)

            vals = plsc.load_gather(allv, [lane * L])
            gm = jnp.max(vals)
            t = plsc.all_reduce_ffs(vals == gm)
            tb = jnp.broadcast_to(t, (L,)).astype(jnp.int32) * L
            wx1 = plsc.load_gather(allv, [tb + 1])
            wy1 = plsc.load_gather(allv, [tb + 2])
            wx2 = plsc.load_gather(allv, [tb + 3])
            wy2 = plsc.load_gather(allv, [tb + 4])

            valid = gm > NEG
            gmv = jnp.full((L,), gm, jnp.float32)
            orow = jnp.where(lane == 0, wx1,
                   jnp.where(lane == 1, wy1,
                   jnp.where(lane == 2, wx2,
                   jnp.where(lane == 3, wy2,
                   jnp.where(lane == 4, gmv, jnp.float32(0.0))))))
            orow = jnp.where((lane < 5) & valid, orow, jnp.float32(0.0))
            outv[pl.ds(i * L, L)] = orow

            return fused_pass(wx1, wy1, wx2, wy2)

        lax.fori_loop(0, MAX_DET, round_body, (m0, li0))

        @pl.when(jnp.logical_and(cid == 0, sid == 0))
        def _():
            pltpu.sync_copy(outv, out_hbm)

    return nms


def kernel(boxes, scores):
    n = boxes.shape[0]
    n_pad = ((n + NT * L - 1) // (NT * L)) * (NT * L)
    x1 = jnp.zeros((n_pad,), jnp.float32).at[:n].set(boxes[:, 0])
    y1 = jnp.zeros((n_pad,), jnp.float32).at[:n].set(boxes[:, 1])
    x2 = jnp.zeros((n_pad,), jnp.float32).at[:n].set(boxes[:, 2])
    y2 = jnp.zeros((n_pad,), jnp.float32).at[:n].set(boxes[:, 3])
    sc = jnp.zeros((n_pad,), jnp.float32).at[:n].set(scores)
    out = _make_nms(n_pad)(x1, y1, x2, y2, sc)
    return out.reshape(MAX_DET, L)[:, :5]


# speedup vs baseline: 1.2726x; 1.0133x over previous
import functools

import jax
import jax.numpy as jnp
from jax import lax
from jax.experimental import pallas as pl
from jax.experimental.pallas import tpu as pltpu
from jax.experimental.pallas import tpu_sc as plsc

SCORE_THRESH = 0.05
NMS_THRESH = 0.5
MAX_DET = 100
L = 16
NT = 16
NEG = float("-inf")


def _make_nms(n_pad: int):
    per = n_pad // NT
    chunks = per // L
    mesh = plsc.VectorSubcoreMesh(core_axis_name="c", subcore_axis_name="s", num_cores=1)

    @functools.partial(
        pl.kernel,
        mesh=mesh,
        compiler_params=pltpu.CompilerParams(needs_layout_passes=False),
        out_type=jax.ShapeDtypeStruct((MAX_DET * L,), jnp.float32),
        scratch_types=[
            pltpu.VMEM((per,), jnp.float32),
            pltpu.VMEM((per,), jnp.float32),
            pltpu.VMEM((per,), jnp.float32),
            pltpu.VMEM((per,), jnp.float32),
            pltpu.VMEM((per,), jnp.float32),
            pltpu.VMEM((per,), jnp.float32),
            pltpu.VMEM((L,), jnp.float32),
            pltpu.VMEM((NT * L,), jnp.float32),
            pltpu.VMEM((MAX_DET * L,), jnp.float32),
            pltpu.VMEM_SHARED((2 * NT * L,), jnp.float32),
            pltpu.SemaphoreType.DMA,
        ],
    )
    def nms(x1_hbm, y1_hbm, x2_hbm, y2_hbm, sc_hbm, out_hbm,
            x1v, y1v, x2v, y2v, sv, areav, candv, allv, outv, shared, sem):
        cid = lax.axis_index("c")
        sid = lax.axis_index("s")
        base = sid * per

        cps = [pltpu.make_async_copy(src.at[pl.ds(base, per)], dst, sem)
               for src, dst in ((x1_hbm, x1v), (y1_hbm, y1v), (x2_hbm, x2v),
                                (y2_hbm, y2v), (sc_hbm, sv))]
        for cp in cps:
            cp.start()
        for cp in cps:
            cp.wait()

        lane = lax.iota(jnp.int32, L)

        @plsc.parallel_loop(0, per, step=L, unroll=4)
        def _prep(o):
            sl = pl.ds(o, L)
            a = (x2v[sl] - x1v[sl]) * (y2v[sl] - y1v[sl])
            areav[sl] = a
            s = sv[sl]
            sv[sl] = jnp.where(s > SCORE_THRESH, s, NEG)

        big = jnp.int32(2 ** 30)

        def fused_pass(wx1, wy1, wx2, wy2):
            warea = (wx2 - wx1) * (wy2 - wy1)

            mv0 = jnp.full((L,), NEG, jnp.float32)
            mi0 = jnp.zeros((L,), jnp.int32)

            @plsc.parallel_loop(0, per, step=L, unroll=3, carry=(mv0, mi0))
            def body(o, carry):
                mv, mi = carry
                sl = pl.ds(o, L)
                bx1 = x1v[sl]
                by1 = y1v[sl]
                bx2 = x2v[sl]
                by2 = y2v[sl]
                ltx = jnp.maximum(bx1, wx1)
                lty = jnp.maximum(by1, wy1)
                rbx = jnp.minimum(bx2, wx2)
                rby = jnp.minimum(by2, wy2)
                w = jnp.maximum(rbx - ltx, 0.0)
                h = jnp.maximum(rby - lty, 0.0)
                inter = w * h
                iou = inter / (areav[sl] + warea - inter + 1e-9)
                ns = jnp.where(iou > NMS_THRESH, NEG, sv[sl])
                sv[sl] = ns
                gt = ns > mv
                mv = jnp.where(gt, ns, mv)
                mi = jnp.where(gt, jnp.full((L,), o, jnp.int32), mi)
                return (mv, mi)

            mv, mi = body
            m = jnp.max(mv)
            cand = jnp.where(mv == m, mi + lane, big)
            li = jnp.min(cand)
            return m, li

        m0, li0 = fused_pass(
            jnp.full((L,), 1e9, jnp.float32),
            jnp.full((L,), 1e9, jnp.float32),
            jnp.full((L,), -1e9, jnp.float32),
            jnp.full((L,), -1e9, jnp.float32),
        )

        zero = jnp.zeros((L,), jnp.int32)

        def round_body(i, carry):
            m, li = carry
            liv = jnp.full((L,), li, jnp.int32)
            cx1 = plsc.load_gather(x1v, [liv])
            cy1 = plsc.load_gather(y1v, [liv])
            cx2 = plsc.load_gather(x2v, [liv])
            cy2 = plsc.load_gather(y2v, [liv])
            mvv = jnp.full((L,), m, jnp.float32)
            row = jnp.where(lane == 0, mvv,
                  jnp.where(lane == 1, cx1,
                  jnp.where(lane == 2, cy1,
                  jnp.where(lane == 3, cx2,
                  jnp.where(lane == 4, cy2, jnp.float32(0.0))))))
            candv[...] = row
            p = lax.rem(i, 2) * (NT * L)
            pltpu.sync_copy(candv, shared.at[pl.ds(p + sid * L, L)])
            plsc.subcore_barrier()
            pltpu.sync_copy(shared.at[pl.ds(p, NT * L)], allv)

            vals = plsc.load_gather(allv, [lane * L])
            gm = jnp.max(vals)
            t = plsc.all_reduce_ffs(vals == gm)
            tb = jnp.broadcast_to(t, (L,)).astype(jnp.int32) * L
            wx1 = plsc.load_gather(allv, [tb + 1])
            wy1 = plsc.load_gather(allv, [tb + 2])
            wx2 = plsc.load_gather(allv, [tb + 3])
            wy2 = plsc.load_gather(allv, [tb + 4])

            valid = gm > NEG
            gmv = jnp.full((L,), gm, jnp.float32)
            orow = jnp.where(lane == 0, wx1,
                   jnp.where(lane == 1, wy1,
                   jnp.where(lane == 2, wx2,
                   jnp.where(lane == 3, wy2,
                   jnp.where(lane == 4, gmv, jnp.float32(0.0))))))
            orow = jnp.where((lane < 5) & valid, orow, jnp.float32(0.0))
            outv[pl.ds(i * L, L)] = orow

            return fused_pass(wx1, wy1, wx2, wy2)

        lax.fori_loop(0, MAX_DET, round_body, (m0, li0))

        @pl.when(jnp.logical_and(cid == 0, sid == 0))
        def _():
            pltpu.sync_copy(outv, out_hbm)

    return nms


def kernel(boxes, scores):
    n = boxes.shape[0]
    n_pad = ((n + NT * L - 1) // (NT * L)) * (NT * L)
    x1 = jnp.zeros((n_pad,), jnp.float32).at[:n].set(boxes[:, 0])
    y1 = jnp.zeros((n_pad,), jnp.float32).at[:n].set(boxes[:, 1])
    x2 = jnp.zeros((n_pad,), jnp.float32).at[:n].set(boxes[:, 2])
    y2 = jnp.zeros((n_pad,), jnp.float32).at[:n].set(boxes[:, 3])
    sc = jnp.zeros((n_pad,), jnp.float32).at[:n].set(scores)
    out = _make_nms(n_pad)(x1, y1, x2, y2, sc)
    return out.reshape(MAX_DET, L)[:, :5]
